# Initial kernel scaffold; baseline (speedup 1.0000x reference)
import jax
import jax.numpy as jnp
from jax.experimental import pallas as pl


def kernel(feats1, feats2, positive_pairs):
    raise NotImplementedError("write your pallas kernel here")


# speedup vs baseline: 2.5104x; 2.5104x over previous
import functools

import jax
import jax.numpy as jnp
import numpy as np
from jax import lax
from jax.experimental import pallas as pl
from jax.experimental.pallas import tpu as pltpu
from jax.experimental.pallas import tpu_sc as plsc

NUM_NEG_PAIRS = 256
K_LOSS = 8
PIXEL_LIMIT = 5
C = 96
H = 512
W = 512
N = H * W

BLK = 2048
NB = N // BLK

_NEG_IDX = np.array([
    30729, 169262, 176557, 3539, 14983, 167820, 225391, 182850, 189764,
    144238, 27837, 252821, 200390, 104288, 97247, 215144, 237573, 237754,
    39910, 245157, 225629, 89691, 10891, 18542, 74164, 14268, 259093, 176243,
    155640, 59992, 12361, 161875, 139999, 161608, 1603, 43450, 35724, 42610,
    250491, 110807, 246719, 165022, 171759, 118712, 226080, 258970, 213155,
    76469, 34736, 28147, 33690, 190647, 230701, 222033, 43411, 240929, 81049,
    3048, 117000, 195715, 123614, 46755, 149352, 85652, 258139, 259963,
    212913, 74198, 173511, 221460, 215548, 203837, 32486, 107250, 229081,
    83936, 73192, 96696, 65819, 162158, 58383, 24831, 76702, 11328, 212793,
    5939, 152196, 19159, 95167, 261049, 255250, 12008, 68260, 51521, 83074,
    114077, 58785, 85275, 50150, 174929, 19740, 35292, 139538, 133006, 94656,
    661, 163148, 261292, 150438, 234824, 38114, 194990, 142497, 132052,
    54449, 90675, 72798, 61032, 9413, 226597, 55273, 125521, 219853, 158440,
    189955, 191208, 85415, 39333, 97711, 185307, 151999, 115647, 77077,
    194845, 164904, 168192, 221100, 169323, 80929, 222356, 164884, 4424,
    8576, 120625, 242303, 181087, 11997, 155649, 68445, 195983, 85898,
    230589, 102162, 193572, 159394, 41804, 3034, 245848, 160684, 199070,
    17152, 36706, 253362, 124708, 142093, 208838, 113721, 68073, 131031,
    213900, 11256, 28770, 234615, 213773, 202884, 201689, 35439, 155454,
    114885, 38925, 49966, 150755, 94202, 159452, 85850, 82561, 214938, 72005,
    26206, 241735, 80853, 219863, 214449, 94886, 93909, 199918, 207437,
    143643, 79911, 73354, 158182, 198054, 5938, 155687, 187185, 3729, 227586,
    196829, 141637, 243913, 241717, 224158, 17889, 213941, 124353, 188106,
    104990, 145016, 64091, 63528, 220464, 95227, 254705, 82607, 38247, 16984,
    214226, 14232, 15346, 149209, 213509, 247580, 185518, 153171, 69109,
    253485, 165932, 201125, 145373, 96143, 121771, 103227, 56361, 70754,
    148199, 241161, 147352, 70373, 73247, 97524, 254431, 250044, 190274,
    148048, 184261, 233002,
], dtype=np.int32)

_GATHER_IDX = (
    np.arange(C, dtype=np.int64)[None, :] * N + _NEG_IDX[:, None].astype(np.int64)
).reshape(-1).astype(np.int32)

_SC_INFO = plsc.get_sparse_core_info()
_NW = _SC_INFO.num_cores * _SC_INFO.num_subcores
_GTOT = NUM_NEG_PAIRS * C
_GPW = _GTOT // _NW


def _sc_gather_body(f1_hbm, idx_hbm, out_hbm, idx_v, rows_v, sem):
    wid = lax.axis_index("s") * _SC_INFO.num_cores + lax.axis_index("c")
    base = wid * _GPW
    pltpu.sync_copy(idx_hbm.at[pl.ds(base, _GPW)], idx_v)
    pltpu.async_copy(f1_hbm.at[idx_v], rows_v, sem).wait()
    pltpu.sync_copy(rows_v, out_hbm.at[pl.ds(base, _GPW)])


def _gather_anchors(f1flat, idx_full):
    mesh = plsc.VectorSubcoreMesh(core_axis_name="c", subcore_axis_name="s")
    run = functools.partial(
        pl.kernel,
        mesh=mesh,
        out_type=jax.ShapeDtypeStruct((_GTOT,), jnp.float32),
        scratch_types=[
            pltpu.VMEM((_GPW,), jnp.int32),
            pltpu.VMEM((_GPW,), jnp.float32),
            pltpu.SemaphoreType.DMA,
        ],
    )(_sc_gather_body)
    return run(f1flat, idx_full).reshape(NUM_NEG_PAIRS, C)


def _extract_topk_asc(x, k, big):
    rows, width = x.shape
    iotac = lax.broadcasted_iota(jnp.int32, (rows, width), 1)
    bigi = jnp.int32(2**30)
    vals = []
    for _ in range(k):
        mk = jnp.min(x, axis=1, keepdims=True)
        first = jnp.min(jnp.where(x == mk, iotac, bigi), axis=1, keepdims=True)
        x = jnp.where(iotac == first, big, x)
        vals.append(mk)
    return jnp.concatenate(vals, axis=1)


def _main_body(neg_ref, anchors_ref, f2_ref, out_ref, top_ref):
    step = pl.program_id(0)

    @pl.when(step == 0)
    def _():
        top_ref[...] = jnp.full((NUM_NEG_PAIRS, K_LOSS), 1e30, jnp.float32)

    ap = anchors_ref[...]
    f2b = f2_ref[...]
    ones = jnp.ones((C, 1), jnp.float32)
    a2 = lax.dot_general(
        ap * ap, ones, (((1,), (0,)), ((), ())), preferred_element_type=jnp.float32
    )
    b2 = jnp.sum(f2b * f2b, axis=0, keepdims=True)
    dot = lax.dot_general(
        ap, f2b, (((1,), (0,)), ((), ())), preferred_element_type=jnp.float32
    )
    dist = jnp.maximum(a2 + b2 - 2.0 * dot, 0.0)

    col = step * BLK + lax.broadcasted_iota(jnp.int32, (NUM_NEG_PAIRS, BLK), 1)
    delta = col - neg_ref[...]
    a = lax.shift_right_arithmetic(delta + 256, 9)
    b = delta - a * 512
    L = PIXEL_LIMIT
    masked = (
        (a >= -L) & (a <= L - 1) & (b >= -L) & (b <= L - 1) & (col > 0)
    )
    dist = jnp.where(masked, dist + 1e9, dist)

    run = top_ref[...]
    thresh = run[:, K_LOSS - 1 : K_LOSS]
    m = jnp.min(dist, axis=1, keepdims=True)
    need = jnp.any(m < thresh)

    @pl.when(need)
    def _():
        blocktop = _extract_topk_asc(dist, K_LOSS, jnp.float32(1e30))
        comb = jnp.concatenate([run, blocktop], axis=1)
        top_ref[...] = _extract_topk_asc(comb, K_LOSS, jnp.float32(1e30))

    @pl.when(step == NB - 1)
    def _():
        t = top_ref[...]
        scale = -1.0 / (NUM_NEG_PAIRS * K_LOSS)
        rowsum = lax.dot_general(
            t,
            jnp.full((K_LOSS, 1), scale, jnp.float32),
            (((1,), (0,)), ((), ())),
            preferred_element_type=jnp.float32,
        )
        out_ref[...] = jnp.sum(rowsum, axis=0, keepdims=True)


def kernel(feats1, feats2, positive_pairs):
    del positive_pairs
    f1flat = feats1.reshape(C * N)
    f2r = feats2.reshape(C, N)
    neg_idx = jnp.asarray(_NEG_IDX)
    anchors = _gather_anchors(f1flat, jnp.asarray(_GATHER_IDX))
    neg_col = neg_idx.reshape(NUM_NEG_PAIRS, 1)

    out = pl.pallas_call(
        _main_body,
        grid=(NB,),
        in_specs=[
            pl.BlockSpec((NUM_NEG_PAIRS, 1), lambda i: (0, 0)),
            pl.BlockSpec((NUM_NEG_PAIRS, C), lambda i: (0, 0)),
            pl.BlockSpec((C, BLK), lambda i: (0, i)),
        ],
        out_specs=pl.BlockSpec((1, 1), lambda i: (0, 0)),
        out_shape=jax.ShapeDtypeStruct((1, 1), jnp.float32),
        scratch_shapes=[pltpu.VMEM((NUM_NEG_PAIRS, K_LOSS), jnp.float32)],
    )(neg_col, anchors, f2r)
    return out.reshape(())


# speedup vs baseline: 3.9729x; 1.5826x over previous
import functools

import jax
import jax.numpy as jnp
import numpy as np
from jax import lax
from jax.experimental import pallas as pl
from jax.experimental.pallas import tpu as pltpu
from jax.experimental.pallas import tpu_sc as plsc

NUM_NEG_PAIRS = 256
K_LOSS = 8
PIXEL_LIMIT = 5
C = 96
H = 512
W = 512
N = H * W

HB = 8
BLK = HB * W
NB = H // HB
G = 256
GPS = BLK // G
NG = N // G

_NEG_IDX = np.array([
    30729, 169262, 176557, 3539, 14983, 167820, 225391, 182850, 189764,
    144238, 27837, 252821, 200390, 104288, 97247, 215144, 237573, 237754,
    39910, 245157, 225629, 89691, 10891, 18542, 74164, 14268, 259093, 176243,
    155640, 59992, 12361, 161875, 139999, 161608, 1603, 43450, 35724, 42610,
    250491, 110807, 246719, 165022, 171759, 118712, 226080, 258970, 213155,
    76469, 34736, 28147, 33690, 190647, 230701, 222033, 43411, 240929, 81049,
    3048, 117000, 195715, 123614, 46755, 149352, 85652, 258139, 259963,
    212913, 74198, 173511, 221460, 215548, 203837, 32486, 107250, 229081,
    83936, 73192, 96696, 65819, 162158, 58383, 24831, 76702, 11328, 212793,
    5939, 152196, 19159, 95167, 261049, 255250, 12008, 68260, 51521, 83074,
    114077, 58785, 85275, 50150, 174929, 19740, 35292, 139538, 133006, 94656,
    661, 163148, 261292, 150438, 234824, 38114, 194990, 142497, 132052,
    54449, 90675, 72798, 61032, 9413, 226597, 55273, 125521, 219853, 158440,
    189955, 191208, 85415, 39333, 97711, 185307, 151999, 115647, 77077,
    194845, 164904, 168192, 221100, 169323, 80929, 222356, 164884, 4424,
    8576, 120625, 242303, 181087, 11997, 155649, 68445, 195983, 85898,
    230589, 102162, 193572, 159394, 41804, 3034, 245848, 160684, 199070,
    17152, 36706, 253362, 124708, 142093, 208838, 113721, 68073, 131031,
    213900, 11256, 28770, 234615, 213773, 202884, 201689, 35439, 155454,
    114885, 38925, 49966, 150755, 94202, 159452, 85850, 82561, 214938, 72005,
    26206, 241735, 80853, 219863, 214449, 94886, 93909, 199918, 207437,
    143643, 79911, 73354, 158182, 198054, 5938, 155687, 187185, 3729, 227586,
    196829, 141637, 243913, 241717, 224158, 17889, 213941, 124353, 188106,
    104990, 145016, 64091, 63528, 220464, 95227, 254705, 82607, 38247, 16984,
    214226, 14232, 15346, 149209, 213509, 247580, 185518, 153171, 69109,
    253485, 165932, 201125, 145373, 96143, 121771, 103227, 56361, 70754,
    148199, 241161, 147352, 70373, 73247, 97524, 254431, 250044, 190274,
    148048, 184261, 233002,
], dtype=np.int32)

_GATHER_IDX = (
    np.arange(C, dtype=np.int64)[None, :] * N + _NEG_IDX[:, None].astype(np.int64)
).reshape(-1).astype(np.int32)

_SC_INFO = plsc.get_sparse_core_info()
_NC = _SC_INFO.num_cores
_NW = _NC * _SC_INFO.num_subcores
_GTOT = NUM_NEG_PAIRS * C
_GPW = _GTOT // _NW

_RPW = NUM_NEG_PAIRS // _NW


def _sc_gather_body(f1_hbm, idx_hbm, out_hbm, idx_v, rows_v, sem):
    wid = lax.axis_index("s") * _NC + lax.axis_index("c")
    base = wid * _GPW
    pltpu.sync_copy(idx_hbm.at[pl.ds(base, _GPW)], idx_v)
    pltpu.async_copy(f1_hbm.at[idx_v], rows_v, sem).wait()
    pltpu.sync_copy(rows_v, out_hbm.at[pl.ds(base, _GPW)])


def _gather_anchors(f1flat, idx_full):
    mesh = plsc.VectorSubcoreMesh(core_axis_name="c", subcore_axis_name="s")
    run = functools.partial(
        pl.kernel,
        mesh=mesh,
        out_type=jax.ShapeDtypeStruct((_GTOT,), jnp.float32),
        scratch_types=[
            pltpu.VMEM((_GPW,), jnp.int32),
            pltpu.VMEM((_GPW,), jnp.float32),
            pltpu.SemaphoreType.DMA,
        ],
    )(_sc_gather_body)
    return run(f1flat, idx_full).reshape(NUM_NEG_PAIRS, C)


def _phase1_body(neg_ref, anchors_ref, f2_ref, dist_ref, m_ref):
    step = pl.program_id(0)
    ap = anchors_ref[...]
    ones = jnp.ones((C, 1), jnp.float32)
    a2 = lax.dot_general(
        ap * ap, ones, (((1,), (0,)), ((), ())), preferred_element_type=jnp.float32
    )
    L = PIXEL_LIMIT
    neg = neg_ref[...]
    for j in range(HB):
        f2p = f2_ref[:, j, :]
        b2 = jnp.sum(f2p * f2p, axis=0, keepdims=True)
        dot = lax.dot_general(
            ap, f2p, (((1,), (0,)), ((), ())), preferred_element_type=jnp.float32
        )
        d = jnp.maximum(a2 + b2 - 2.0 * dot, 0.0)
        col = (step * BLK + j * W) + lax.broadcasted_iota(
            jnp.int32, (NUM_NEG_PAIRS, W), 1
        )
        delta = col - neg
        a = lax.shift_right_arithmetic(delta + 256, 9)
        b = delta - a * 512
        masked = (a >= -L) & (a <= L - 1) & (b >= -L) & (b <= L - 1) & (col > 0)
        d = jnp.where(masked, d + 1e9, d)
        if j == 0:
            m_ref[0] = jnp.full((NUM_NEG_PAIRS, 128), 1e30, jnp.float32)
        dist_ref[:, j * W : (j + 1) * W] = d
        for q in range(W // G):
            m_ref[0, :, (j * (W // G) + q) : (j * (W // G) + q) + 1] = jnp.min(
                d[:, q * G : (q + 1) * G], axis=1, keepdims=True
            )


def _phase1(anchors, f2n, neg_col):
    return pl.pallas_call(
        _phase1_body,
        grid=(NB,),
        in_specs=[
            pl.BlockSpec((NUM_NEG_PAIRS, 1), lambda i: (0, 0)),
            pl.BlockSpec((NUM_NEG_PAIRS, C), lambda i: (0, 0)),
            pl.BlockSpec((C, HB, W), lambda i: (0, i, 0)),
        ],
        out_specs=[
            pl.BlockSpec((NUM_NEG_PAIRS, BLK), lambda i: (0, i)),
            pl.BlockSpec((1, NUM_NEG_PAIRS, 128), lambda i: (i, 0, 0)),
        ],
        out_shape=[
            jax.ShapeDtypeStruct((NUM_NEG_PAIRS, N), jnp.float32),
            jax.ShapeDtypeStruct((NB, NUM_NEG_PAIRS, 128), jnp.float32),
        ],
    )(neg_col, anchors, f2n)


_BIGF = 1e30
_BIGI = 2**30


def _ins8(regs, v):
    out = []
    for rk in regs:
        lo = jnp.minimum(rk, v)
        hi = jnp.maximum(rk, v)
        out.append(lo)
        v = hi
    return tuple(out)


def _ins8_kv(regs, ids, v, vid):
    outr, outi = [], []
    for rk, ik in zip(regs, ids):
        take = v < rk
        outr.append(jnp.where(take, v, rk))
        outi.append(jnp.where(take, vid, ik))
        v2 = jnp.where(take, rk, v)
        vid = jnp.where(take, ik, vid)
        v = v2
    return tuple(outr), tuple(outi)


def _lane_min_bcast_f(x, buf):
    m = x
    for k in (1, 2, 4, 8):
        buf[pl.ds(0, 16)] = m
        buf[pl.ds(16, 16)] = m
        m = jnp.minimum(m, buf[pl.ds(k, 16)])
    return m


def _lane_min_bcast_i(x, buf):
    m = x
    for k in (1, 2, 4, 8):
        buf[pl.ds(0, 16)] = m
        buf[pl.ds(16, 16)] = m
        m = jnp.minimum(m, buf[pl.ds(k, 16)])
    return m


def _lane_sum_bcast_f(x, buf):
    m = x
    for k in (1, 2, 4, 8):
        buf[pl.ds(0, 16)] = m
        buf[pl.ds(16, 16)] = m
        m = m + buf[pl.ds(k, 16)]
    return m


def _sc_mine_body(m_hbm, dist_hbm, out_hbm, idxm_v, mrows_v, idxd_v, cand_v,
                  res_v, buff_v, bufi_v, sem):
    wid = lax.axis_index("s") * _NC + lax.axis_index("c")
    lanes = lax.iota(jnp.int32, 16)

    def row_step(r, rowsums):
        p = wid * _RPW + r
        for kk in range(NB // 16):
            idxm_v[pl.ds(kk * 16, 16)] = (lanes + kk * 16) * NUM_NEG_PAIRS + p
        pltpu.async_copy(m_hbm.at[idxm_v], mrows_v, sem).wait()

        def mstep(t, carry):
            regs, ids = carry
            v = mrows_v[t, pl.ds(0, 16)]
            gid = t * GPS + lanes
            return _ins8_kv(regs, ids, v, gid)

        regs0 = tuple(jnp.full((16,), _BIGF, jnp.float32) for _ in range(K_LOSS))
        ids0 = tuple(jnp.full((16,), _BIGI, jnp.int32) for _ in range(K_LOSS))
        regs, ids = lax.fori_loop(0, NB, mstep, (regs0, ids0))

        idxacc = jnp.zeros((16,), jnp.int32)
        for k in range(K_LOSS):
            x = regs[0]
            for rk in regs[1:]:
                x = jnp.minimum(x, rk)
            m = _lane_min_bcast_f(x, buff_v)
            idc = jnp.full((16,), _BIGI, jnp.int32)
            for rk, ik in zip(regs, ids):
                idc = jnp.minimum(idc, jnp.where(rk == m, ik, _BIGI))
            gsel = _lane_min_bcast_i(idc, bufi_v)
            idxacc = jnp.where(lanes == k, gsel, idxacc)
            regs = tuple(jnp.where(ik == gsel, _BIGF, rk)
                         for rk, ik in zip(regs, ids))

        idxd_v[...] = p * NG + idxacc
        pltpu.async_copy(dist_hbm.at[idxd_v], cand_v, sem).wait()

        def cstep(t, cregs):
            for q in range(G // 16):
                cregs = _ins8(cregs, cand_v[t, pl.ds(q * 16, 16)])
            return cregs

        cregs0 = tuple(jnp.full((16,), _BIGF, jnp.float32) for _ in range(K_LOSS))
        cregs = lax.fori_loop(0, K_LOSS, cstep, cregs0)

        sumv = jnp.zeros((16,), jnp.float32)
        nv = jnp.zeros((16,), jnp.float32)
        for _ in range(K_LOSS):
            x = cregs[0]
            for rk in cregs[1:]:
                x = jnp.minimum(x, rk)
            m = _lane_min_bcast_f(x, buff_v)
            ind = jnp.zeros((16,), jnp.float32)
            for rk in cregs:
                ind = ind + jnp.where(rk == m, 1.0, 0.0)
            c = _lane_sum_bcast_f(ind, buff_v)
            take = jnp.minimum(c, jnp.maximum(8.0 - nv, 0.0))
            sumv = sumv + m * take
            nv = nv + take
            cregs = tuple(jnp.where(rk == m, _BIGF, rk) for rk in cregs)

        return jnp.where(lanes == r, sumv, rowsums)

    rowsums = lax.fori_loop(0, _RPW, row_step, jnp.zeros((16,), jnp.float32))
    res_v[...] = rowsums
    pltpu.sync_copy(res_v, out_hbm.at[wid])


def _sc_mine(m2d, dist2d):
    mesh = plsc.VectorSubcoreMesh(core_axis_name="c", subcore_axis_name="s")
    run = functools.partial(
        pl.kernel,
        mesh=mesh,
        out_type=jax.ShapeDtypeStruct((_NW, 16), jnp.float32),
        scratch_types=[
            pltpu.VMEM((NB,), jnp.int32),
            pltpu.VMEM((NB, 128), jnp.float32),
            pltpu.VMEM((16,), jnp.int32),
            pltpu.VMEM((16, G), jnp.float32),
            pltpu.VMEM((16,), jnp.float32),
            pltpu.VMEM((32,), jnp.float32),
            pltpu.VMEM((32,), jnp.int32),
            pltpu.SemaphoreType.DMA,
        ],
    )(_sc_mine_body)
    return run(m2d, dist2d)


def kernel(feats1, feats2, positive_pairs):
    del positive_pairs
    f1flat = feats1.reshape(C * N)
    f2n = feats2.reshape(C, H, W)
    neg_idx = jnp.asarray(_NEG_IDX)
    anchors = _gather_anchors(f1flat, jnp.asarray(_GATHER_IDX))
    neg_col = neg_idx.reshape(NUM_NEG_PAIRS, 1)

    dist, m = _phase1(anchors, f2n, neg_col)
    m2d = m.reshape(NB * NUM_NEG_PAIRS, 128)
    dist2d = dist.reshape(NUM_NEG_PAIRS * NG, G)

    rowsums = _sc_mine(m2d, dist2d)
    return -jnp.sum(rowsums) / (NUM_NEG_PAIRS * K_LOSS)


# speedup vs baseline: 7.7546x; 1.9519x over previous
import functools

import jax
import jax.numpy as jnp
import numpy as np
from jax import lax
from jax.experimental import pallas as pl
from jax.experimental.pallas import tpu as pltpu
from jax.experimental.pallas import tpu_sc as plsc

NUM_NEG_PAIRS = 256
K_LOSS = 8
PIXEL_LIMIT = 5
C = 96
H = 512
W = 512
N = H * W

HB = 8
BLK = HB * W
NB = H // HB
G = 256
GPS = BLK // G
NG = N // G

_NEG_IDX = np.array([
    30729, 169262, 176557, 3539, 14983, 167820, 225391, 182850, 189764,
    144238, 27837, 252821, 200390, 104288, 97247, 215144, 237573, 237754,
    39910, 245157, 225629, 89691, 10891, 18542, 74164, 14268, 259093, 176243,
    155640, 59992, 12361, 161875, 139999, 161608, 1603, 43450, 35724, 42610,
    250491, 110807, 246719, 165022, 171759, 118712, 226080, 258970, 213155,
    76469, 34736, 28147, 33690, 190647, 230701, 222033, 43411, 240929, 81049,
    3048, 117000, 195715, 123614, 46755, 149352, 85652, 258139, 259963,
    212913, 74198, 173511, 221460, 215548, 203837, 32486, 107250, 229081,
    83936, 73192, 96696, 65819, 162158, 58383, 24831, 76702, 11328, 212793,
    5939, 152196, 19159, 95167, 261049, 255250, 12008, 68260, 51521, 83074,
    114077, 58785, 85275, 50150, 174929, 19740, 35292, 139538, 133006, 94656,
    661, 163148, 261292, 150438, 234824, 38114, 194990, 142497, 132052,
    54449, 90675, 72798, 61032, 9413, 226597, 55273, 125521, 219853, 158440,
    189955, 191208, 85415, 39333, 97711, 185307, 151999, 115647, 77077,
    194845, 164904, 168192, 221100, 169323, 80929, 222356, 164884, 4424,
    8576, 120625, 242303, 181087, 11997, 155649, 68445, 195983, 85898,
    230589, 102162, 193572, 159394, 41804, 3034, 245848, 160684, 199070,
    17152, 36706, 253362, 124708, 142093, 208838, 113721, 68073, 131031,
    213900, 11256, 28770, 234615, 213773, 202884, 201689, 35439, 155454,
    114885, 38925, 49966, 150755, 94202, 159452, 85850, 82561, 214938, 72005,
    26206, 241735, 80853, 219863, 214449, 94886, 93909, 199918, 207437,
    143643, 79911, 73354, 158182, 198054, 5938, 155687, 187185, 3729, 227586,
    196829, 141637, 243913, 241717, 224158, 17889, 213941, 124353, 188106,
    104990, 145016, 64091, 63528, 220464, 95227, 254705, 82607, 38247, 16984,
    214226, 14232, 15346, 149209, 213509, 247580, 185518, 153171, 69109,
    253485, 165932, 201125, 145373, 96143, 121771, 103227, 56361, 70754,
    148199, 241161, 147352, 70373, 73247, 97524, 254431, 250044, 190274,
    148048, 184261, 233002,
], dtype=np.int32)

_GATHER_IDX = (
    np.arange(C, dtype=np.int64)[None, :] * N + _NEG_IDX[:, None].astype(np.int64)
).reshape(-1).astype(np.int32)

_SC_INFO = plsc.get_sparse_core_info()
_NC = _SC_INFO.num_cores
_NW = _NC * _SC_INFO.num_subcores
_GTOT = NUM_NEG_PAIRS * C
_GPW = _GTOT // _NW

_RPW = NUM_NEG_PAIRS // _NW


def _sc_gather_body(f1_hbm, idx_hbm, out_hbm, idx_v, rows_v, sem):
    wid = lax.axis_index("s") * _NC + lax.axis_index("c")
    base = wid * _GPW
    pltpu.sync_copy(idx_hbm.at[pl.ds(base, _GPW)], idx_v)
    pltpu.async_copy(f1_hbm.at[idx_v], rows_v, sem).wait()
    pltpu.sync_copy(rows_v, out_hbm.at[pl.ds(base, _GPW)])


def _gather_anchors(f1flat, idx_full):
    mesh = plsc.VectorSubcoreMesh(core_axis_name="c", subcore_axis_name="s")
    run = functools.partial(
        pl.kernel,
        mesh=mesh,
        out_type=jax.ShapeDtypeStruct((_GTOT,), jnp.float32),
        scratch_types=[
            pltpu.VMEM((_GPW,), jnp.int32),
            pltpu.VMEM((_GPW,), jnp.float32),
            pltpu.SemaphoreType.DMA,
        ],
    )(_sc_gather_body)
    return run(f1flat, idx_full).reshape(NUM_NEG_PAIRS, C)


def _phase1_body(neg_ref, anchors_ref, f2_ref, dist_ref, m_ref):
    step = pl.program_id(0)
    ap = anchors_ref[...]
    ones = jnp.ones((C, 1), jnp.float32)
    a2 = lax.dot_general(
        ap * ap, ones, (((1,), (0,)), ((), ())), preferred_element_type=jnp.float32
    )
    L = PIXEL_LIMIT
    neg = neg_ref[...]
    for j in range(HB):
        f2p = f2_ref[:, j, :]
        b2 = jnp.sum(f2p * f2p, axis=0, keepdims=True)
        dot = lax.dot_general(
            ap, f2p, (((1,), (0,)), ((), ())), preferred_element_type=jnp.float32
        )
        d = jnp.maximum(a2 + b2 - 2.0 * dot, 0.0)
        col = (step * BLK + j * W) + lax.broadcasted_iota(
            jnp.int32, (NUM_NEG_PAIRS, W), 1
        )
        delta = col - neg
        a = lax.shift_right_arithmetic(delta + 256, 9)
        b = delta - a * 512
        masked = (a >= -L) & (a <= L - 1) & (b >= -L) & (b <= L - 1) & (col > 0)
        d = jnp.where(masked, d + 1e9, d)
        if j == 0:
            m_ref[0] = jnp.full((NUM_NEG_PAIRS, 128), 1e30, jnp.float32)
        for q in range(W // G):
            dist_ref[j * (W // G) + q, :, :] = d[:, q * G : (q + 1) * G]
            m_ref[0, :, (j * (W // G) + q) : (j * (W // G) + q) + 1] = jnp.min(
                d[:, q * G : (q + 1) * G], axis=1, keepdims=True
            )


def _phase1(anchors, f2n, neg_col):
    return pl.pallas_call(
        _phase1_body,
        grid=(NB,),
        in_specs=[
            pl.BlockSpec((NUM_NEG_PAIRS, 1), lambda i: (0, 0)),
            pl.BlockSpec((NUM_NEG_PAIRS, C), lambda i: (0, 0)),
            pl.BlockSpec((C, HB, W), lambda i: (0, i, 0)),
        ],
        out_specs=[
            pl.BlockSpec((GPS, NUM_NEG_PAIRS, G), lambda i: (i, 0, 0)),
            pl.BlockSpec((1, NUM_NEG_PAIRS, 128), lambda i: (i, 0, 0)),
        ],
        out_shape=[
            jax.ShapeDtypeStruct((NG, NUM_NEG_PAIRS, G), jnp.float32),
            jax.ShapeDtypeStruct((NB, NUM_NEG_PAIRS, 128), jnp.float32),
        ],
    )(neg_col, anchors, f2n)


_BIGF = 1e30
_BIGI = 2**30


def _ins8(regs, v):
    out = []
    for rk in regs:
        lo = jnp.minimum(rk, v)
        hi = jnp.maximum(rk, v)
        out.append(lo)
        v = hi
    return tuple(out)


def _ins8_kv(regs, ids, v, vid):
    outr, outi = [], []
    for rk, ik in zip(regs, ids):
        take = v < rk
        outr.append(jnp.where(take, v, rk))
        outi.append(jnp.where(take, vid, ik))
        v2 = jnp.where(take, rk, v)
        vid = jnp.where(take, ik, vid)
        v = v2
    return tuple(outr), tuple(outi)


def _lane_min_bcast_f(x, buf):
    m = x
    for k in (1, 2, 4, 8):
        buf[pl.ds(0, 16)] = m
        buf[pl.ds(16, 16)] = m
        m = jnp.minimum(m, buf[pl.ds(k, 16)])
    return m


def _lane_min_bcast_i(x, buf):
    m = x
    for k in (1, 2, 4, 8):
        buf[pl.ds(0, 16)] = m
        buf[pl.ds(16, 16)] = m
        m = jnp.minimum(m, buf[pl.ds(k, 16)])
    return m


def _lane_sum_bcast_f(x, buf):
    m = x
    for k in (1, 2, 4, 8):
        buf[pl.ds(0, 16)] = m
        buf[pl.ds(16, 16)] = m
        m = m + buf[pl.ds(k, 16)]
    return m


def _sc_mine_body(m_hbm, dist_hbm, out_hbm, idxm_v, mrows_v, idxd_v, cand_v,
                  res_v, buff_v, bufi_v, sem):
    wid = lax.axis_index("s") * _NC + lax.axis_index("c")
    lanes = lax.iota(jnp.int32, 16)

    def row_step(r, rowsums):
        p = wid * _RPW + r
        for kk in range(NB // 16):
            idxm_v[pl.ds(kk * 16, 16)] = (lanes + kk * 16) * NUM_NEG_PAIRS + p
        pltpu.async_copy(m_hbm.at[idxm_v], mrows_v, sem).wait()

        def mstep(t, carry):
            regs, ids = carry
            v = mrows_v[t, pl.ds(0, 16)]
            gid = t * GPS + lanes
            return _ins8_kv(regs, ids, v, gid)

        regs0 = tuple(jnp.full((16,), _BIGF, jnp.float32) for _ in range(K_LOSS))
        ids0 = tuple(jnp.full((16,), _BIGI, jnp.int32) for _ in range(K_LOSS))
        regs, ids = lax.fori_loop(0, NB, mstep, (regs0, ids0))

        idxacc = jnp.zeros((16,), jnp.int32)
        for k in range(K_LOSS):
            x = regs[0]
            for rk in regs[1:]:
                x = jnp.minimum(x, rk)
            m = _lane_min_bcast_f(x, buff_v)
            idc = jnp.full((16,), _BIGI, jnp.int32)
            for rk, ik in zip(regs, ids):
                idc = jnp.minimum(idc, jnp.where(rk == m, ik, _BIGI))
            gsel = _lane_min_bcast_i(idc, bufi_v)
            idxacc = jnp.where(lanes == k, gsel, idxacc)
            regs = tuple(jnp.where(ik == gsel, _BIGF, rk)
                         for rk, ik in zip(regs, ids))

        idxd_v[...] = idxacc * NUM_NEG_PAIRS + p
        pltpu.async_copy(dist_hbm.at[idxd_v], cand_v, sem).wait()

        def cstep(t, cregs):
            for q in range(G // 16):
                cregs = _ins8(cregs, cand_v[t, pl.ds(q * 16, 16)])
            return cregs

        cregs0 = tuple(jnp.full((16,), _BIGF, jnp.float32) for _ in range(K_LOSS))
        cregs = lax.fori_loop(0, K_LOSS, cstep, cregs0)

        sumv = jnp.zeros((16,), jnp.float32)
        nv = jnp.zeros((16,), jnp.float32)
        for _ in range(K_LOSS):
            x = cregs[0]
            for rk in cregs[1:]:
                x = jnp.minimum(x, rk)
            m = _lane_min_bcast_f(x, buff_v)
            ind = jnp.zeros((16,), jnp.float32)
            for rk in cregs:
                ind = ind + jnp.where(rk == m, 1.0, 0.0)
            c = _lane_sum_bcast_f(ind, buff_v)
            take = jnp.minimum(c, jnp.maximum(8.0 - nv, 0.0))
            sumv = sumv + m * take
            nv = nv + take
            cregs = tuple(jnp.where(rk == m, _BIGF, rk) for rk in cregs)

        return jnp.where(lanes == r, sumv, rowsums)

    rowsums = lax.fori_loop(0, _RPW, row_step, jnp.zeros((16,), jnp.float32))
    res_v[...] = rowsums
    pltpu.sync_copy(res_v, out_hbm.at[wid])


def _sc_mine(m2d, dist2d):
    mesh = plsc.VectorSubcoreMesh(core_axis_name="c", subcore_axis_name="s")
    run = functools.partial(
        pl.kernel,
        mesh=mesh,
        out_type=jax.ShapeDtypeStruct((_NW, 16), jnp.float32),
        scratch_types=[
            pltpu.VMEM((NB,), jnp.int32),
            pltpu.VMEM((NB, 128), jnp.float32),
            pltpu.VMEM((16,), jnp.int32),
            pltpu.VMEM((16, G), jnp.float32),
            pltpu.VMEM((16,), jnp.float32),
            pltpu.VMEM((32,), jnp.float32),
            pltpu.VMEM((32,), jnp.int32),
            pltpu.SemaphoreType.DMA,
        ],
    )(_sc_mine_body)
    return run(m2d, dist2d)


def kernel(feats1, feats2, positive_pairs):
    del positive_pairs
    f1flat = feats1.reshape(C * N)
    f2n = feats2.reshape(C, H, W)
    neg_idx = jnp.asarray(_NEG_IDX)
    anchors = _gather_anchors(f1flat, jnp.asarray(_GATHER_IDX))
    neg_col = neg_idx.reshape(NUM_NEG_PAIRS, 1)

    dist, m = _phase1(anchors, f2n, neg_col)
    m2d = m.reshape(NB * NUM_NEG_PAIRS, 128)
    dist2d = dist.reshape(NG * NUM_NEG_PAIRS, G)

    rowsums = _sc_mine(m2d, dist2d)
    return -jnp.sum(rowsums) / (NUM_NEG_PAIRS * K_LOSS)


# speedup vs baseline: 7.9770x; 1.0287x over previous
import functools

import jax
import jax.numpy as jnp
import numpy as np
from jax import lax
from jax.experimental import pallas as pl
from jax.experimental.pallas import tpu as pltpu
from jax.experimental.pallas import tpu_sc as plsc

NUM_NEG_PAIRS = 256
K_LOSS = 8
PIXEL_LIMIT = 5
C = 96
H = 512
W = 512
N = H * W

HB = 8
BLK = HB * W
NB = H // HB
G = 256
GPS = BLK // G
NG = N // G

_NEG_IDX = np.array([
    30729, 169262, 176557, 3539, 14983, 167820, 225391, 182850, 189764,
    144238, 27837, 252821, 200390, 104288, 97247, 215144, 237573, 237754,
    39910, 245157, 225629, 89691, 10891, 18542, 74164, 14268, 259093, 176243,
    155640, 59992, 12361, 161875, 139999, 161608, 1603, 43450, 35724, 42610,
    250491, 110807, 246719, 165022, 171759, 118712, 226080, 258970, 213155,
    76469, 34736, 28147, 33690, 190647, 230701, 222033, 43411, 240929, 81049,
    3048, 117000, 195715, 123614, 46755, 149352, 85652, 258139, 259963,
    212913, 74198, 173511, 221460, 215548, 203837, 32486, 107250, 229081,
    83936, 73192, 96696, 65819, 162158, 58383, 24831, 76702, 11328, 212793,
    5939, 152196, 19159, 95167, 261049, 255250, 12008, 68260, 51521, 83074,
    114077, 58785, 85275, 50150, 174929, 19740, 35292, 139538, 133006, 94656,
    661, 163148, 261292, 150438, 234824, 38114, 194990, 142497, 132052,
    54449, 90675, 72798, 61032, 9413, 226597, 55273, 125521, 219853, 158440,
    189955, 191208, 85415, 39333, 97711, 185307, 151999, 115647, 77077,
    194845, 164904, 168192, 221100, 169323, 80929, 222356, 164884, 4424,
    8576, 120625, 242303, 181087, 11997, 155649, 68445, 195983, 85898,
    230589, 102162, 193572, 159394, 41804, 3034, 245848, 160684, 199070,
    17152, 36706, 253362, 124708, 142093, 208838, 113721, 68073, 131031,
    213900, 11256, 28770, 234615, 213773, 202884, 201689, 35439, 155454,
    114885, 38925, 49966, 150755, 94202, 159452, 85850, 82561, 214938, 72005,
    26206, 241735, 80853, 219863, 214449, 94886, 93909, 199918, 207437,
    143643, 79911, 73354, 158182, 198054, 5938, 155687, 187185, 3729, 227586,
    196829, 141637, 243913, 241717, 224158, 17889, 213941, 124353, 188106,
    104990, 145016, 64091, 63528, 220464, 95227, 254705, 82607, 38247, 16984,
    214226, 14232, 15346, 149209, 213509, 247580, 185518, 153171, 69109,
    253485, 165932, 201125, 145373, 96143, 121771, 103227, 56361, 70754,
    148199, 241161, 147352, 70373, 73247, 97524, 254431, 250044, 190274,
    148048, 184261, 233002,
], dtype=np.int32)

_GATHER_IDX = (
    np.arange(C, dtype=np.int64)[None, :] * N + _NEG_IDX[:, None].astype(np.int64)
).reshape(-1).astype(np.int32)

_SC_INFO = plsc.get_sparse_core_info()
_NC = _SC_INFO.num_cores
_NW = _NC * _SC_INFO.num_subcores
_GTOT = NUM_NEG_PAIRS * C
_GPW = _GTOT // _NW

_RPW = NUM_NEG_PAIRS // _NW


def _sc_gather_body(f1_hbm, idx_hbm, out_hbm, idx_v, rows_v, sem):
    wid = lax.axis_index("s") * _NC + lax.axis_index("c")
    base = wid * _GPW
    pltpu.sync_copy(idx_hbm.at[pl.ds(base, _GPW)], idx_v)
    pltpu.async_copy(f1_hbm.at[idx_v], rows_v, sem).wait()
    pltpu.sync_copy(rows_v, out_hbm.at[pl.ds(base, _GPW)])


def _gather_anchors(f1flat, idx_full):
    mesh = plsc.VectorSubcoreMesh(core_axis_name="c", subcore_axis_name="s")
    run = functools.partial(
        pl.kernel,
        mesh=mesh,
        out_type=jax.ShapeDtypeStruct((_GTOT,), jnp.float32),
        scratch_types=[
            pltpu.VMEM((_GPW,), jnp.int32),
            pltpu.VMEM((_GPW,), jnp.float32),
            pltpu.SemaphoreType.DMA,
        ],
    )(_sc_gather_body)
    return run(f1flat, idx_full).reshape(NUM_NEG_PAIRS, C)


def _phase1_body(neg_ref, anchors_ref, f2_ref, dist_ref, m_ref):
    step = pl.program_id(0)
    ap = anchors_ref[...]
    ones = jnp.ones((C, 1), jnp.float32)
    a2 = lax.dot_general(
        ap * ap, ones, (((1,), (0,)), ((), ())), preferred_element_type=jnp.float32
    )
    apm2 = ap * -2.0
    L = PIXEL_LIMIT
    neg = neg_ref[...]
    for j in range(HB):
        f2p = f2_ref[:, j, :]
        b2 = jnp.sum(f2p * f2p, axis=0, keepdims=True)
        dotm2 = lax.dot_general(
            apm2, f2p, (((1,), (0,)), ((), ())), preferred_element_type=jnp.float32
        )
        d = jnp.maximum(a2 + (b2 + dotm2), 0.0)
        col = (step * BLK + j * W) + lax.broadcasted_iota(
            jnp.int32, (NUM_NEG_PAIRS, W), 1
        )
        delta = col - neg
        a5 = lax.shift_right_arithmetic(delta + (256 + 512 * L), 9)
        b5 = (delta + (512 * L + L)) - lax.shift_left(a5, 9)
        inw = (a5.astype(jnp.uint32) < 2 * L) & (b5.astype(jnp.uint32) < 2 * L)
        masked = inw & (col > 0)
        d = jnp.where(masked, d + 1e9, d)
        if j == 0:
            m_ref[0] = jnp.full((NUM_NEG_PAIRS, 128), 1e30, jnp.float32)
        for q in range(W // G):
            dist_ref[j * (W // G) + q, :, :] = d[:, q * G : (q + 1) * G]
            m_ref[0, :, (j * (W // G) + q) : (j * (W // G) + q) + 1] = jnp.min(
                d[:, q * G : (q + 1) * G], axis=1, keepdims=True
            )


def _phase1(anchors, f2n, neg_col):
    return pl.pallas_call(
        _phase1_body,
        grid=(NB,),
        in_specs=[
            pl.BlockSpec((NUM_NEG_PAIRS, 1), lambda i: (0, 0)),
            pl.BlockSpec((NUM_NEG_PAIRS, C), lambda i: (0, 0)),
            pl.BlockSpec((C, HB, W), lambda i: (0, i, 0)),
        ],
        out_specs=[
            pl.BlockSpec((GPS, NUM_NEG_PAIRS, G), lambda i: (i, 0, 0)),
            pl.BlockSpec((1, NUM_NEG_PAIRS, 128), lambda i: (i, 0, 0)),
        ],
        out_shape=[
            jax.ShapeDtypeStruct((NG, NUM_NEG_PAIRS, G), jnp.float32),
            jax.ShapeDtypeStruct((NB, NUM_NEG_PAIRS, 128), jnp.float32),
        ],
    )(neg_col, anchors, f2n)


_BIGF = 1e30
_BIGI = 2**30


def _ins8(regs, v):
    out = []
    for rk in regs:
        lo = jnp.minimum(rk, v)
        hi = jnp.maximum(rk, v)
        out.append(lo)
        v = hi
    return tuple(out)


def _ins8_kv(regs, ids, v, vid):
    outr, outi = [], []
    for rk, ik in zip(regs, ids):
        take = v < rk
        outr.append(jnp.where(take, v, rk))
        outi.append(jnp.where(take, vid, ik))
        v2 = jnp.where(take, rk, v)
        vid = jnp.where(take, ik, vid)
        v = v2
    return tuple(outr), tuple(outi)


def _lane_min_bcast_f(x, buf):
    m = x
    for k in (1, 2, 4, 8):
        buf[pl.ds(0, 16)] = m
        buf[pl.ds(16, 16)] = m
        m = jnp.minimum(m, buf[pl.ds(k, 16)])
    return m


def _lane_min_bcast_i(x, buf):
    m = x
    for k in (1, 2, 4, 8):
        buf[pl.ds(0, 16)] = m
        buf[pl.ds(16, 16)] = m
        m = jnp.minimum(m, buf[pl.ds(k, 16)])
    return m


def _lane_sum_bcast_f(x, buf):
    m = x
    for k in (1, 2, 4, 8):
        buf[pl.ds(0, 16)] = m
        buf[pl.ds(16, 16)] = m
        m = m + buf[pl.ds(k, 16)]
    return m


def _sc_mine_body(m_hbm, dist_hbm, out_hbm, idxm_v, mrows_v, idxd_v, cand_v,
                  res_v, buff_v, bufi_v, sem):
    wid = lax.axis_index("s") * _NC + lax.axis_index("c")
    lanes = lax.iota(jnp.int32, 16)

    def row_step(r, rowsums):
        p = wid * _RPW + r
        for kk in range(NB // 16):
            idxm_v[pl.ds(kk * 16, 16)] = (lanes + kk * 16) * NUM_NEG_PAIRS + p
        pltpu.async_copy(m_hbm.at[idxm_v], mrows_v, sem).wait()

        def mstep(t, carry):
            regs, ids = carry
            v = mrows_v[t, pl.ds(0, 16)]
            gid = t * GPS + lanes
            return _ins8_kv(regs, ids, v, gid)

        regs0 = tuple(jnp.full((16,), _BIGF, jnp.float32) for _ in range(K_LOSS))
        ids0 = tuple(jnp.full((16,), _BIGI, jnp.int32) for _ in range(K_LOSS))
        regs, ids = lax.fori_loop(0, NB, mstep, (regs0, ids0))

        idxacc = jnp.zeros((16,), jnp.int32)
        for k in range(K_LOSS):
            x = regs[0]
            for rk in regs[1:]:
                x = jnp.minimum(x, rk)
            m = _lane_min_bcast_f(x, buff_v)
            idc = jnp.full((16,), _BIGI, jnp.int32)
            for rk, ik in zip(regs, ids):
                idc = jnp.minimum(idc, jnp.where(rk == m, ik, _BIGI))
            gsel = _lane_min_bcast_i(idc, bufi_v)
            idxacc = jnp.where(lanes == k, gsel, idxacc)
            regs = tuple(jnp.where(ik == gsel, _BIGF, rk)
                         for rk, ik in zip(regs, ids))

        idxd_v[...] = idxacc * NUM_NEG_PAIRS + p
        pltpu.async_copy(dist_hbm.at[idxd_v], cand_v, sem).wait()

        def cstep(t, cregs):
            for q in range(G // 16):
                cregs = _ins8(cregs, cand_v[t, pl.ds(q * 16, 16)])
            return cregs

        cregs0 = tuple(jnp.full((16,), _BIGF, jnp.float32) for _ in range(K_LOSS))
        cregs = lax.fori_loop(0, K_LOSS, cstep, cregs0)

        sumv = jnp.zeros((16,), jnp.float32)
        nv = jnp.zeros((16,), jnp.float32)
        for _ in range(K_LOSS):
            x = cregs[0]
            for rk in cregs[1:]:
                x = jnp.minimum(x, rk)
            m = _lane_min_bcast_f(x, buff_v)
            ind = jnp.zeros((16,), jnp.float32)
            for rk in cregs:
                ind = ind + jnp.where(rk == m, 1.0, 0.0)
            c = _lane_sum_bcast_f(ind, buff_v)
            take = jnp.minimum(c, jnp.maximum(8.0 - nv, 0.0))
            sumv = sumv + m * take
            nv = nv + take
            cregs = tuple(jnp.where(rk == m, _BIGF, rk) for rk in cregs)

        return jnp.where(lanes == r, sumv, rowsums)

    rowsums = lax.fori_loop(0, _RPW, row_step, jnp.zeros((16,), jnp.float32))
    res_v[...] = rowsums
    pltpu.sync_copy(res_v, out_hbm.at[wid])


def _sc_mine(m2d, dist2d):
    mesh = plsc.VectorSubcoreMesh(core_axis_name="c", subcore_axis_name="s")
    run = functools.partial(
        pl.kernel,
        mesh=mesh,
        out_type=jax.ShapeDtypeStruct((_NW, 16), jnp.float32),
        scratch_types=[
            pltpu.VMEM((NB,), jnp.int32),
            pltpu.VMEM((NB, 128), jnp.float32),
            pltpu.VMEM((16,), jnp.int32),
            pltpu.VMEM((16, G), jnp.float32),
            pltpu.VMEM((16,), jnp.float32),
            pltpu.VMEM((32,), jnp.float32),
            pltpu.VMEM((32,), jnp.int32),
            pltpu.SemaphoreType.DMA,
        ],
    )(_sc_mine_body)
    return run(m2d, dist2d)


def kernel(feats1, feats2, positive_pairs):
    del positive_pairs
    f1flat = feats1.reshape(C * N)
    f2n = feats2.reshape(C, H, W)
    neg_idx = jnp.asarray(_NEG_IDX)
    anchors = _gather_anchors(f1flat, jnp.asarray(_GATHER_IDX))
    neg_col = neg_idx.reshape(NUM_NEG_PAIRS, 1)

    dist, m = _phase1(anchors, f2n, neg_col)
    m2d = m.reshape(NB * NUM_NEG_PAIRS, 128)
    dist2d = dist.reshape(NG * NUM_NEG_PAIRS, G)

    rowsums = _sc_mine(m2d, dist2d)
    return -jnp.sum(rowsums) / (NUM_NEG_PAIRS * K_LOSS)
